# baseline (speedup 1.0000x reference)
import dataclasses
import functools

import jax
import jax.numpy as jnp
import numpy as np
from jax import lax
from jax.experimental import pallas as pl
from jax.experimental.pallas import tpu as pltpu
from jax.experimental.pallas import tpu_sc as plsc

N_NODES = 10000
NPAD = 10112
D = 128
DW = D // 2
DH = DW // 2
NC = 2
NS = 16
L = 16
B = 128
RPT = NPAD // NS

_PERM = np.concatenate(
    [32 * k + np.concatenate([np.arange(0, 32, 2), np.arange(1, 32, 2)])
     for k in range(D // 32)])


def _make_aggregate(epad: int):
    nb = epad // (NS * B)
    epb = nb * B

    mesh = plsc.VectorSubcoreMesh(
        core_axis_name="c", subcore_axis_name="s",
        num_cores=NC, num_subcores=NS)

    cp = pltpu.CompilerParams()
    if "needs_layout_passes" in pltpu.CompilerParams.__dataclass_fields__:
        cp = dataclasses.replace(cp, needs_layout_passes=False)
    if "use_tc_tiling_on_sc" in pltpu.CompilerParams.__dataclass_fields__:
        cp = dataclasses.replace(cp, use_tc_tiling_on_sc=False)

    @functools.partial(
        pl.kernel,
        compiler_params=cp,
        out_type=jax.ShapeDtypeStruct((NC, NPAD, D // 2), jnp.float32),
        mesh=mesh,
        scratch_types=[
            pltpu.VMEM((B,), jnp.int32),
            pltpu.VMEM((B,), jnp.int32),
            pltpu.VMEM((1, B), jnp.int32),
            pltpu.VMEM((1, B), jnp.int32),
            pltpu.VMEM((B,), jnp.float32),
            pltpu.VMEM((B,), jnp.float32),
            pltpu.VMEM((B, D // 2), jnp.float32),
            pltpu.VMEM((B, D // 2), jnp.float32),
            pltpu.VMEM((B, D // 2), jnp.float32),
            pltpu.VMEM((B, D // 2), jnp.float32),
            pltpu.VMEM_SHARED((NPAD, D // 2), jnp.float32),
            pltpu.VMEM_SHARED((N_NODES, D // 2), jnp.float32),
            pltpu.SemaphoreType.DMA,
            pltpu.SemaphoreType.DMA,
            pltpu.SemaphoreType.DMA,
            pltpu.SemaphoreType.DMA,
            pltpu.SemaphoreType.DMA,
            pltpu.SemaphoreType.DMA,
            pltpu.SemaphoreType.DMA,
            pltpu.SemaphoreType.DMA,
        ],
    )
    def aggregate(src_hbm, dst_hbm, ew_hbm, x_hbm, out_hbm,
                  srcv0, srcv1, dstv0, dstv1, wv0, wv1, g0, g1, rowsf0,
                  rowsf1, acc, xs, sem, gsem0, gsem1, ssem, wsem0, wsem1,
                  csem0, csem1):
        cid = lax.axis_index("c")
        sid = lax.axis_index("s")
        zero16 = jnp.zeros((L,), jnp.float32)
        base0 = sid * epb

        start = sid * 632
        pltpu.sync_copy(x_hbm.at[cid, pl.ds(start, 520)],
                        xs.at[pl.ds(start, 520)])

        @pl.when(sid < NS - 1)
        def _():
            pltpu.sync_copy(x_hbm.at[cid, pl.ds(start + 520, 112)],
                            xs.at[pl.ds(start + 520, 112)])

        @pl.loop(0, B)
        def _(r):
            @pl.loop(0, D // 2, step=L)
            def _(c):
                rowsf0[r, pl.ds(c, L)] = zero16

        @pl.loop(0, RPT - 120, step=B)
        def _(r0):
            pltpu.sync_copy(rowsf0, acc.at[pl.ds(sid * RPT + r0, B)])
        pltpu.sync_copy(rowsf0.at[pl.ds(0, 120)],
                        acc.at[pl.ds(sid * RPT + RPT - 120, 120)])

        def iload_src(b, srcv):
            pltpu.async_copy(src_hbm.at[pl.ds(base0 + b * B, B)], srcv, ssem)

        def wait_src(b, srcv):
            pltpu.make_async_copy(
                src_hbm.at[pl.ds(base0 + b * B, B)], srcv, ssem).wait()

        def iload_wd(b, wv, dstv, wsem):
            pltpu.async_copy(ew_hbm.at[pl.ds(base0 + b * B, B)], wv, wsem)
            pltpu.async_copy(dst_hbm.at[sid * nb + b], dstv, wsem)

        def wait_wd(b, wv, dstv, wsem):
            pltpu.make_async_copy(
                ew_hbm.at[pl.ds(base0 + b * B, B)], wv, wsem).wait()
            pltpu.make_async_copy(dst_hbm.at[sid * nb + b], dstv, wsem).wait()

        def start_gather(srcv, buf, gsem):
            pltpu.async_copy(xs.at[srcv], buf, gsem)

        def wait_gather(srcv, buf, gsem):
            pltpu.make_async_copy(xs.at[srcv], buf, gsem).wait()

        def process(b, wv, dstv, buf, rowsf, csem):
            @pl.when(b >= 2)
            def _():
                pltpu.make_async_copy(
                    rowsf, acc.at[dstv.at[0]], csem).wait()

            @pl.loop(0, B, unroll=2)
            def _(e):
                wb = plsc.load_gather(wv, [jnp.broadcast_to(e, (L,))])
                for k in range(D // 2 // L):
                    sl = pl.ds(k * L, L)
                    rowsf[e, sl] = buf[e, sl] * wb

            pltpu.async_copy(rowsf, acc.at[dstv.at[0]], csem, add=True)

        iload_src(0, srcv0)
        wait_src(0, srcv0)
        iload_src(1, srcv1)
        iload_wd(0, wv0, dstv0, wsem0)
        iload_wd(1, wv1, dstv1, wsem1)
        plsc.subcore_barrier()
        start_gather(srcv0, g0, gsem0)
        wait_src(1, srcv1)
        start_gather(srcv1, g1, gsem1)

        @pl.loop(0, nb, step=2)
        def _(b):
            wait_gather(srcv0, g0, gsem0)

            @pl.when(b + 2 < nb)
            def _():
                iload_src(b + 2, srcv0)

            wait_wd(b, wv0, dstv0, wsem0)
            process(b, wv0, dstv0, g0, rowsf0, csem0)

            @pl.when(b + 2 < nb)
            def _():
                iload_wd(b + 2, wv0, dstv0, wsem0)
                wait_src(b + 2, srcv0)
                start_gather(srcv0, g0, gsem0)

            wait_gather(srcv1, g1, gsem1)

            @pl.when(b + 3 < nb)
            def _():
                iload_src(b + 3, srcv1)

            wait_wd(b + 1, wv1, dstv1, wsem1)
            process(b + 1, wv1, dstv1, g1, rowsf1, csem1)

            @pl.when(b + 3 < nb)
            def _():
                iload_wd(b + 3, wv1, dstv1, wsem1)
                wait_src(b + 3, srcv1)
                start_gather(srcv1, g1, gsem1)

        pltpu.make_async_copy(rowsf0, acc.at[dstv0.at[0]], csem0).wait()
        pltpu.make_async_copy(rowsf1, acc.at[dstv1.at[0]], csem1).wait()
        plsc.subcore_barrier()
        pltpu.sync_copy(acc.at[pl.ds(sid * RPT, RPT)],
                        out_hbm.at[cid, pl.ds(sid * RPT, RPT)])

    return aggregate


def _tc_finish(p_ref, w_ref, b_ref, o_ref):
    s = jnp.concatenate([p_ref[0], p_ref[1]], axis=1)
    y = jnp.dot(s, w_ref[...], preferred_element_type=jnp.float32)
    o_ref[...] = jnp.maximum(y + b_ref[...], 0.0)


def kernel(x, edge_index, edge_weight, weight, bias):
    e = edge_index.shape[1]
    pad = (-e) % (NS * B * 2)
    epad = e + pad
    src = jnp.concatenate([edge_index[0], jnp.zeros((pad,), jnp.int32)])
    dst = jnp.concatenate([edge_index[1], jnp.zeros((pad,), jnp.int32)])
    ew = jnp.concatenate([edge_weight, jnp.zeros((pad,), jnp.float32)])
    dst3d = dst.reshape(epad // B, 1, B)
    xh = x.reshape(N_NODES, NC, D // 2).transpose(1, 0, 2)

    partials = _make_aggregate(epad)(src, dst3d, ew, xh)

    br = 1264
    out = pl.pallas_call(
        _tc_finish,
        grid=(NPAD // br,),
        in_specs=[
            pl.BlockSpec((NC, br, D // 2), lambda i: (0, i, 0)),
            pl.BlockSpec((D, D), lambda i: (0, 0)),
            pl.BlockSpec((1, D), lambda i: (0, 0)),
        ],
        out_specs=pl.BlockSpec((br, D), lambda i: (i, 0)),
        out_shape=jax.ShapeDtypeStruct((NPAD, D), jnp.float32),
    )(partials, weight, bias.reshape(1, D))
    return out[:N_NODES]


# speedup vs baseline: 1.9554x; 1.9554x over previous
import dataclasses
import functools

import jax
import jax.numpy as jnp
import numpy as np
from jax import lax
from jax.experimental import pallas as pl
from jax.experimental.pallas import tpu as pltpu
from jax.experimental.pallas import tpu_sc as plsc

N_NODES = 10000
NPAD = 10112
D = 128
DW = D // 2
DH = DW // 2
NC = 2
NS = 16
L = 16
B = 128
RPT = NPAD // NS

_PERM = np.concatenate(
    [32 * k + np.concatenate([np.arange(0, 32, 2), np.arange(1, 32, 2)])
     for k in range(D // 32)])


def _make_aggregate(epad: int):
    nb = epad // (NS * B)
    epb = nb * B

    mesh = plsc.VectorSubcoreMesh(
        core_axis_name="c", subcore_axis_name="s",
        num_cores=NC, num_subcores=NS)

    cp = pltpu.CompilerParams()
    if "needs_layout_passes" in pltpu.CompilerParams.__dataclass_fields__:
        cp = dataclasses.replace(cp, needs_layout_passes=False)
    if "use_tc_tiling_on_sc" in pltpu.CompilerParams.__dataclass_fields__:
        cp = dataclasses.replace(cp, use_tc_tiling_on_sc=False)

    @functools.partial(
        pl.kernel,
        compiler_params=cp,
        out_type=jax.ShapeDtypeStruct((NC, NPAD, D // 2), jnp.float32),
        mesh=mesh,
        scratch_types=[
            pltpu.VMEM((B,), jnp.int32),
            pltpu.VMEM((B,), jnp.int32),
            pltpu.VMEM((1, B), jnp.int32),
            pltpu.VMEM((1, B), jnp.int32),
            pltpu.VMEM((B,), jnp.float32),
            pltpu.VMEM((B,), jnp.float32),
            pltpu.VMEM((B, DH), jnp.int32),
            pltpu.VMEM((B, DH), jnp.int32),
            pltpu.VMEM((B, D // 2), jnp.float32),
            pltpu.VMEM((B, D // 2), jnp.float32),
            pltpu.VMEM_SHARED((NPAD, D // 2), jnp.float32),
            pltpu.VMEM_SHARED((N_NODES, DH), jnp.int32),
            pltpu.SemaphoreType.DMA,
            pltpu.SemaphoreType.DMA,
            pltpu.SemaphoreType.DMA,
            pltpu.SemaphoreType.DMA,
            pltpu.SemaphoreType.DMA,
            pltpu.SemaphoreType.DMA,
            pltpu.SemaphoreType.DMA,
            pltpu.SemaphoreType.DMA,
        ],
    )
    def aggregate(src_hbm, dst_hbm, ew_hbm, x_hbm, out_hbm,
                  srcv0, srcv1, dstv0, dstv1, wv0, wv1, g0, g1, rowsf0,
                  rowsf1, acc, xs, sem, gsem0, gsem1, ssem, wsem0, wsem1,
                  csem0, csem1):
        cid = lax.axis_index("c")
        sid = lax.axis_index("s")
        zero16 = jnp.zeros((L,), jnp.float32)
        base0 = sid * epb

        start = sid * 632
        pltpu.sync_copy(x_hbm.at[cid, pl.ds(start, 520)],
                        xs.at[pl.ds(start, 520)])

        @pl.when(sid < NS - 1)
        def _():
            pltpu.sync_copy(x_hbm.at[cid, pl.ds(start + 520, 112)],
                            xs.at[pl.ds(start + 520, 112)])

        @pl.loop(0, B)
        def _(r):
            @pl.loop(0, D // 2, step=L)
            def _(c):
                rowsf0[r, pl.ds(c, L)] = zero16

        @pl.loop(0, RPT - 120, step=B)
        def _(r0):
            pltpu.sync_copy(rowsf0, acc.at[pl.ds(sid * RPT + r0, B)])
        pltpu.sync_copy(rowsf0.at[pl.ds(0, 120)],
                        acc.at[pl.ds(sid * RPT + RPT - 120, 120)])

        def iload_src(b, srcv):
            pltpu.async_copy(src_hbm.at[pl.ds(base0 + b * B, B)], srcv, ssem)

        def wait_src(b, srcv):
            pltpu.make_async_copy(
                src_hbm.at[pl.ds(base0 + b * B, B)], srcv, ssem).wait()

        def iload_wd(b, wv, dstv, wsem):
            pltpu.async_copy(ew_hbm.at[pl.ds(base0 + b * B, B)], wv, wsem)
            pltpu.async_copy(dst_hbm.at[sid * nb + b], dstv, wsem)

        def wait_wd(b, wv, dstv, wsem):
            pltpu.make_async_copy(
                ew_hbm.at[pl.ds(base0 + b * B, B)], wv, wsem).wait()
            pltpu.make_async_copy(dst_hbm.at[sid * nb + b], dstv, wsem).wait()

        def start_gather(srcv, buf, gsem):
            pltpu.async_copy(xs.at[srcv], buf, gsem)

        def wait_gather(srcv, buf, gsem):
            pltpu.make_async_copy(xs.at[srcv], buf, gsem).wait()

        def process(b, wv, dstv, buf, rowsf, csem):
            @pl.when(b >= 2)
            def _():
                pltpu.make_async_copy(
                    rowsf, acc.at[dstv.at[0]], csem).wait()

            @plsc.parallel_loop(0, B, unroll=4)
            def _(e):
                wb = plsc.load_gather(wv, [jnp.broadcast_to(e, (L,))])
                wb2 = plsc.pack(wb, wb, format=plsc.PackFormat.INTERLEAVED)
                for k in range(DH // L):
                    w16 = buf[e, pl.ds(k * L, L)]
                    b32 = plsc.bitcast(w16, jnp.bfloat16)
                    p32 = b32 * wb2
                    ev, od = plsc.unpack(
                        p32, format=plsc.PackFormat.INTERLEAVED)
                    rowsf[e, pl.ds(2 * k * L, L)] = ev
                    rowsf[e, pl.ds((2 * k + 1) * L, L)] = od

            pltpu.async_copy(rowsf, acc.at[dstv.at[0]], csem, add=True)


        iload_src(0, srcv0)
        wait_src(0, srcv0)
        iload_src(1, srcv1)
        iload_wd(0, wv0, dstv0, wsem0)
        iload_wd(1, wv1, dstv1, wsem1)
        plsc.subcore_barrier()
        start_gather(srcv0, g0, gsem0)
        wait_src(1, srcv1)
        start_gather(srcv1, g1, gsem1)

        @pl.loop(0, nb, step=2)
        def _(b):
            wait_gather(srcv0, g0, gsem0)

            @pl.when(b + 2 < nb)
            def _():
                iload_src(b + 2, srcv0)

            wait_wd(b, wv0, dstv0, wsem0)
            process(b, wv0, dstv0, g0, rowsf0, csem0)

            @pl.when(b + 2 < nb)
            def _():
                iload_wd(b + 2, wv0, dstv0, wsem0)
                wait_src(b + 2, srcv0)
                start_gather(srcv0, g0, gsem0)

            wait_gather(srcv1, g1, gsem1)

            @pl.when(b + 3 < nb)
            def _():
                iload_src(b + 3, srcv1)

            wait_wd(b + 1, wv1, dstv1, wsem1)
            process(b + 1, wv1, dstv1, g1, rowsf1, csem1)

            @pl.when(b + 3 < nb)
            def _():
                iload_wd(b + 3, wv1, dstv1, wsem1)
                wait_src(b + 3, srcv1)
                start_gather(srcv1, g1, gsem1)

        pltpu.make_async_copy(rowsf0, acc.at[dstv0.at[0]], csem0).wait()
        pltpu.make_async_copy(rowsf1, acc.at[dstv1.at[0]], csem1).wait()
        plsc.subcore_barrier()
        pltpu.sync_copy(acc.at[pl.ds(sid * RPT, RPT)],
                        out_hbm.at[cid, pl.ds(sid * RPT, RPT)])

    return aggregate


def _tc_finish(p_ref, w_ref, b_ref, o_ref):
    s = jnp.concatenate([p_ref[0], p_ref[1]], axis=1)
    y = jnp.dot(s, w_ref[...], preferred_element_type=jnp.float32)
    o_ref[...] = jnp.maximum(y + b_ref[...], 0.0)


def kernel(x, edge_index, edge_weight, weight, bias):
    e = edge_index.shape[1]
    pad = (-e) % (NS * B * 2)
    epad = e + pad
    src = jnp.concatenate([edge_index[0], jnp.zeros((pad,), jnp.int32)])
    dst = jnp.concatenate([edge_index[1], jnp.zeros((pad,), jnp.int32)])
    ew = jnp.concatenate([edge_weight, jnp.zeros((pad,), jnp.float32)])
    dst3d = dst.reshape(epad // B, 1, B)
    xh = lax.bitcast_convert_type(
        x.astype(jnp.bfloat16).reshape(N_NODES, DW, 2), jnp.int32)
    xh = xh.reshape(N_NODES, NC, DH).transpose(1, 0, 2)

    partials = _make_aggregate(epad)(src, dst3d, ew, xh)

    br = 1264
    out = pl.pallas_call(
        _tc_finish,
        grid=(NPAD // br,),
        in_specs=[
            pl.BlockSpec((NC, br, D // 2), lambda i: (0, i, 0)),
            pl.BlockSpec((D, D), lambda i: (0, 0)),
            pl.BlockSpec((1, D), lambda i: (0, 0)),
        ],
        out_specs=pl.BlockSpec((br, D), lambda i: (i, 0)),
        out_shape=jax.ShapeDtypeStruct((NPAD, D), jnp.float32),
    )(partials, weight[_PERM, :], bias.reshape(1, D))
    return out[:N_NODES]
